# baseline (speedup 1.0000x reference)
import functools

import jax
import jax.numpy as jnp
from jax import lax
from jax.experimental import pallas as pl
from jax.experimental.pallas import tpu as pltpu
from jax.experimental.pallas import tpu_sc as plsc

N_NODES = 10000
N_EDGES = 320000
D_FEAT = 128

NC = 2
NS = 16
NW = NC * NS
EPW = N_EDGES // NW
ECHUNK = 40
NCHUNK = EPW // ECHUNK
NQUAD = (NCHUNK - 2) // 4
ZROWS = ECHUNK
NZBLK = N_NODES // ZROWS
WROWS = 200
NWBLK = N_NODES // WROWS


def _sc_body(dst_hbm, src_hbm, a_hbm, x_hbm, out_hbm,
             src_all, a_all, d0, d1, d2, d3, g0, g1, s0, s1,
             acc, isem, gsem0, gsem1, ssem0, ssem1):
    c = lax.axis_index("c")
    s = lax.axis_index("s")
    wid = c * NS + s
    base = wid * EPW

    dbuf = (d0, d1, d2, d3)
    gbuf = (g0, g1)
    sbuf = (s0, s1)
    gsem = (gsem0, gsem1)
    ssem = (ssem0, ssem1)

    h1 = pltpu.async_copy(src_hbm.at[pl.ds(base, EPW)], src_all, isem)
    h2 = pltpu.async_copy(a_hbm.at[pl.ds(base, EPW)], a_all, isem)

    def zrow(i, carry):
        for j in range(D_FEAT // 16):
            g0[i, pl.ds(j * 16, 16)] = jnp.zeros((16,), jnp.float32)
        return carry
    lax.fori_loop(0, ZROWS, zrow, 0)
    for b in range((NZBLK + NS - 1) // NS):
        blk = b * NS + s

        @pl.when(blk < NZBLK)
        def _():
            pltpu.sync_copy(g0, acc.at[pl.ds(blk * ZROWS, ZROWS)])
    h1.wait()
    h2.wait()
    plsc.subcore_barrier()

    def start_chunk(ci, q4):
        p = q4 % 2
        pltpu.async_copy(dst_hbm.at[pl.ds(base + ci * ECHUNK, ECHUNK)],
                         dbuf[q4], gsem[p])
        pltpu.async_copy(x_hbm.at[src_all.at[pl.ds(ci * ECHUNK, ECHUNK)]],
                         gbuf[p], gsem[p])

    def wait_chunk(q4):
        p = q4 % 2
        pltpu.make_async_copy(dst_hbm.at[pl.ds(0, ECHUNK)],
                              dbuf[q4], gsem[p]).wait()
        pltpu.make_async_copy(x_hbm.at[src_all.at[pl.ds(0, ECHUNK)]],
                              gbuf[p], gsem[p]).wait()

    def scale(ci, q4):
        p = q4 % 2
        g_r, s_r = gbuf[p], sbuf[p]

        def do_edge(e, a):
            for j in range(D_FEAT // 16):
                sl = pl.ds(j * 16, 16)
                s_r[e, sl] = g_r[e, sl] * a

        for grp in range(2):
            av16 = a_all[pl.ds(ci * ECHUNK + grp * 16, 16)]
            for l in range(16):
                do_edge(grp * 16 + l, av16[l])
        av16 = a_all[pl.ds(ci * ECHUNK + 24, 16)]
        for l in range(8, 16):
            do_edge(24 + l, av16[l])

    def start_scatter(q4):
        p = q4 % 2
        pltpu.async_copy(sbuf[p], acc.at[dbuf[q4]], ssem[p], add=True)

    def drain_scatter(q4):
        p = q4 % 2
        pltpu.make_async_copy(sbuf[p], acc.at[dbuf[q4]], ssem[p]).wait()

    start_chunk(0, 0)
    start_chunk(1, 1)

    def quad_body(m, carry):
        i0 = 4 * m
        for b in range(4):
            q4 = b
            qd = (b + 2) % 4
            if b < 2:
                @pl.when(m > 0)
                def _():
                    drain_scatter(qd)
            wait_chunk(q4)
            if b >= 2:
                drain_scatter(qd)
            scale(i0 + b, q4)
            start_scatter(q4)
            start_chunk(i0 + b + 2, qd)
        return carry
    lax.fori_loop(0, NQUAD, quad_body, 0)
    wait_chunk(0)
    drain_scatter(2)
    scale(NCHUNK - 2, 0)
    start_scatter(0)
    wait_chunk(1)
    drain_scatter(3)
    scale(NCHUNK - 1, 1)
    start_scatter(1)
    drain_scatter(0)
    drain_scatter(1)

    plsc.subcore_barrier()
    for b in range((NWBLK + NS - 1) // NS):
        blk = b * NS + s

        @pl.when(blk < NWBLK)
        def _():
            r = blk * WROWS
            pltpu.sync_copy(acc.at[pl.ds(r, WROWS)],
                            out_hbm.at[c, pl.ds(r, WROWS)])


def _combine_body(p_ref, o_ref):
    o_ref[...] = p_ref[0] + p_ref[1]


def kernel(edge_index, A_values, X):
    mesh = plsc.VectorSubcoreMesh(core_axis_name="c", subcore_axis_name="s")
    sc_call = functools.partial(
        pl.kernel,
        mesh=mesh,
        out_type=jax.ShapeDtypeStruct((NC, N_NODES, D_FEAT), jnp.float32),
        scratch_types=[
            pltpu.VMEM((EPW,), jnp.int32),
            pltpu.VMEM((EPW,), jnp.float32),
            pltpu.VMEM((ECHUNK,), jnp.int32),
            pltpu.VMEM((ECHUNK,), jnp.int32),
            pltpu.VMEM((ECHUNK,), jnp.int32),
            pltpu.VMEM((ECHUNK,), jnp.int32),
            pltpu.VMEM((ECHUNK, D_FEAT), jnp.float32),
            pltpu.VMEM((ECHUNK, D_FEAT), jnp.float32),
            pltpu.VMEM((ECHUNK, D_FEAT), jnp.float32),
            pltpu.VMEM((ECHUNK, D_FEAT), jnp.float32),
            pltpu.VMEM_SHARED((N_NODES, D_FEAT), jnp.float32),
            pltpu.SemaphoreType.DMA,
            pltpu.SemaphoreType.DMA,
            pltpu.SemaphoreType.DMA,
            pltpu.SemaphoreType.DMA,
            pltpu.SemaphoreType.DMA,
        ],
    )(_sc_body)
    partials = sc_call(edge_index[0], edge_index[1], A_values, X)

    combine = pl.pallas_call(
        _combine_body,
        out_shape=jax.ShapeDtypeStruct((N_NODES, D_FEAT), jnp.float32),
        grid=(10,),
        in_specs=[pl.BlockSpec((NC, N_NODES // 10, D_FEAT), lambda i: (0, i, 0))],
        out_specs=pl.BlockSpec((N_NODES // 10, D_FEAT), lambda i: (i, 0)),
    )
    return combine(partials)


# speedup vs baseline: 1.0840x; 1.0840x over previous
import functools

import jax
import jax.numpy as jnp
from jax import lax
from jax.experimental import pallas as pl
from jax.experimental.pallas import tpu as pltpu
from jax.experimental.pallas import tpu_sc as plsc

N_NODES = 10000
N_EDGES = 320000
D_FEAT = 128

NC = 2
NS = 16
NW = NC * NS
EPW = N_EDGES // NW
ECHUNK = 80
NCHUNK = EPW // ECHUNK
NQUAD = (NCHUNK - 1) // 4
ZROWS = ECHUNK
NZBLK = N_NODES // ZROWS
WROWS = 200
NWBLK = N_NODES // WROWS


def _sc_body(dst_hbm, src_hbm, a_hbm, x_hbm, out_hbm,
             d0, d1, d2, d3, sr0, sr1, sr2, sr3, a0, a1, a2, a3,
             g0, g1, s0, s1, acc, isem0, isem1, gsem0, gsem1, ssem0, ssem1):
    c = lax.axis_index("c")
    s = lax.axis_index("s")
    wid = c * NS + s
    base = wid * EPW

    dbuf = (d0, d1, d2, d3)
    srbuf = (sr0, sr1, sr2, sr3)
    abuf = (a0, a1, a2, a3)
    gbuf = (g0, g1)
    sbuf = (s0, s1)
    isem = (isem0, isem1)
    gsem = (gsem0, gsem1)
    ssem = (ssem0, ssem1)

    def zrow(i, carry):
        for j in range(D_FEAT // 16):
            g0[i, pl.ds(j * 16, 16)] = jnp.zeros((16,), jnp.float32)
        return carry
    lax.fori_loop(0, ZROWS, zrow, 0)
    for b in range((NZBLK + NS - 1) // NS):
        blk = b * NS + s

        @pl.when(blk < NZBLK)
        def _():
            pltpu.sync_copy(g0, acc.at[pl.ds(blk * ZROWS, ZROWS)])
    plsc.subcore_barrier()

    def fetch_sa(ci, q4):
        p = q4 % 2
        pltpu.async_copy(src_hbm.at[pl.ds(base + ci * ECHUNK, ECHUNK)],
                         srbuf[q4], isem[p])
        pltpu.async_copy(a_hbm.at[pl.ds(base + ci * ECHUNK, ECHUNK)],
                         abuf[q4], isem[p])

    def fetch_dst(ci, q4):
        pltpu.async_copy(dst_hbm.at[pl.ds(base + ci * ECHUNK, ECHUNK)],
                         dbuf[q4], isem[q4 % 2])

    def wait_idx(q4):
        p = q4 % 2
        pltpu.make_async_copy(src_hbm.at[pl.ds(0, ECHUNK)],
                              srbuf[q4], isem[p]).wait()
        pltpu.make_async_copy(a_hbm.at[pl.ds(0, ECHUNK)],
                              abuf[q4], isem[p]).wait()
        pltpu.make_async_copy(dst_hbm.at[pl.ds(0, ECHUNK)],
                              dbuf[q4], isem[p]).wait()

    def start_gather(q4):
        p = q4 % 2
        pltpu.async_copy(x_hbm.at[srbuf[q4]], gbuf[p], gsem[p])

    def wait_gather(q4):
        p = q4 % 2
        pltpu.make_async_copy(x_hbm.at[srbuf[q4]], gbuf[p], gsem[p]).wait()

    def scale(q4):
        p = q4 % 2
        g_r, s_r, a_r = gbuf[p], sbuf[p], abuf[q4]
        for grp in range(ECHUNK // 16):
            av16 = a_r[pl.ds(grp * 16, 16)]
            for l in range(16):
                e = grp * 16 + l
                a = av16[l]
                for j in range(D_FEAT // 16):
                    sl = pl.ds(j * 16, 16)
                    s_r[e, sl] = g_r[e, sl] * a

    def start_scatter(q4):
        p = q4 % 2
        pltpu.async_copy(sbuf[p], acc.at[dbuf[q4]], ssem[p], add=True)

    def drain_scatter(q4):
        p = q4 % 2
        pltpu.make_async_copy(sbuf[p], acc.at[dbuf[q4]], ssem[p]).wait()

    fetch_sa(0, 0)
    fetch_dst(0, 0)
    fetch_sa(1, 1)
    fetch_dst(1, 1)
    wait_idx(0)
    start_gather(0)
    fetch_sa(2, 2)
    wait_idx(1)
    start_gather(1)
    fetch_sa(3, 3)

    def chunk_step(i, q4, first_pair, last_fetch_ok, prefetch_ok):
        if not first_pair:
            drain_scatter((q4 + 2) % 4)

        @pl.when(prefetch_ok)
        def _():
            fetch_dst(i + 2, (q4 + 2) % 4)
        wait_gather(q4)
        scale(q4)
        start_scatter(q4)

        @pl.when(prefetch_ok)
        def _():
            wait_idx((q4 + 2) % 4)

        @pl.when(last_fetch_ok)
        def _():
            fetch_sa(i + 4, q4)

        @pl.when(prefetch_ok)
        def _():
            start_gather((q4 + 2) % 4)

    def quad_body(m, carry):
        i0 = 4 * m
        for b in range(4):
            i = i0 + b
            if b < 2:
                @pl.when(m > 0)
                def _():
                    drain_scatter((b + 2) % 4)
                chunk_step(i, b, True, i + 4 < NCHUNK, i + 2 < NCHUNK)
            else:
                chunk_step(i, b, False, i + 4 < NCHUNK, i + 2 < NCHUNK)
        return carry
    lax.fori_loop(0, NQUAD, quad_body, 0)
    drain_scatter(2)
    wait_gather(0)
    scale(0)
    start_scatter(0)
    drain_scatter(3)
    drain_scatter(0)

    plsc.subcore_barrier()
    for b in range((NWBLK + NS - 1) // NS):
        blk = b * NS + s

        @pl.when(blk < NWBLK)
        def _():
            r = blk * WROWS
            pltpu.sync_copy(acc.at[pl.ds(r, WROWS)],
                            out_hbm.at[c, pl.ds(r, WROWS)])


def _combine_body(p_ref, o_ref):
    o_ref[...] = p_ref[0] + p_ref[1]


def kernel(edge_index, A_values, X):
    mesh = plsc.VectorSubcoreMesh(core_axis_name="c", subcore_axis_name="s")
    sc_call = functools.partial(
        pl.kernel,
        mesh=mesh,
        out_type=jax.ShapeDtypeStruct((NC, N_NODES, D_FEAT), jnp.float32),
        scratch_types=[
            pltpu.VMEM((ECHUNK,), jnp.int32),
            pltpu.VMEM((ECHUNK,), jnp.int32),
            pltpu.VMEM((ECHUNK,), jnp.int32),
            pltpu.VMEM((ECHUNK,), jnp.int32),
            pltpu.VMEM((ECHUNK,), jnp.int32),
            pltpu.VMEM((ECHUNK,), jnp.int32),
            pltpu.VMEM((ECHUNK,), jnp.int32),
            pltpu.VMEM((ECHUNK,), jnp.int32),
            pltpu.VMEM((ECHUNK,), jnp.float32),
            pltpu.VMEM((ECHUNK,), jnp.float32),
            pltpu.VMEM((ECHUNK,), jnp.float32),
            pltpu.VMEM((ECHUNK,), jnp.float32),
            pltpu.VMEM((ECHUNK, D_FEAT), jnp.float32),
            pltpu.VMEM((ECHUNK, D_FEAT), jnp.float32),
            pltpu.VMEM((ECHUNK, D_FEAT), jnp.float32),
            pltpu.VMEM((ECHUNK, D_FEAT), jnp.float32),
            pltpu.VMEM_SHARED((N_NODES, D_FEAT), jnp.float32),
            pltpu.SemaphoreType.DMA,
            pltpu.SemaphoreType.DMA,
            pltpu.SemaphoreType.DMA,
            pltpu.SemaphoreType.DMA,
            pltpu.SemaphoreType.DMA,
            pltpu.SemaphoreType.DMA,
        ],
    )(_sc_body)
    partials = sc_call(edge_index[0], edge_index[1], A_values, X)

    combine = pl.pallas_call(
        _combine_body,
        out_shape=jax.ShapeDtypeStruct((N_NODES, D_FEAT), jnp.float32),
        grid=(10,),
        in_specs=[pl.BlockSpec((NC, N_NODES // 10, D_FEAT), lambda i: (0, i, 0))],
        out_specs=pl.BlockSpec((N_NODES // 10, D_FEAT), lambda i: (i, 0)),
    )
    return combine(partials)
